# baseline (speedup 1.0000x reference)
import dataclasses

import jax
import jax.numpy as jnp
from jax import lax
from jax.experimental import pallas as pl
from jax.experimental.pallas import tpu as pltpu
from jax.experimental.pallas import tpu_sc as plsc

N_NODES_ = 10000
N_EDGES_ = 320000
D_ = 128

NC_ = 2
NS_ = 16
L_ = 16
NW_ = NC_ * NS_
PER_W_ = N_EDGES_ // NW_
W_ = 80
NSLOT_ = 5
NCHUNK_ = PER_W_ // W_
NBATCH_ = W_ // L_
D32_ = D_ // 2

assert (NCHUNK_ - NSLOT_) % NSLOT_ == 0


def _issue_gathers(h_hbm, sidx, didx, u_buf, v_buf, sem, chunk):
  off = pl.multiple_of(chunk * W_, 8)
  pltpu.make_async_copy(h_hbm.at[sidx.at[pl.ds(off, W_)]], u_buf, sem).start()
  pltpu.make_async_copy(h_hbm.at[didx.at[pl.ds(off, W_)]], v_buf, sem).start()


def _wait_gathers(h_hbm, sidx, didx, u_buf, v_buf, sem, chunk):
  off = pl.multiple_of(chunk * W_, 8)
  pltpu.make_async_copy(h_hbm.at[sidx.at[pl.ds(off, W_)]], u_buf, sem).wait()
  pltpu.make_async_copy(h_hbm.at[didx.at[pl.ds(off, W_)]], v_buf, sem).wait()


def _compute_chunk(u_buf, v_buf, acc, scores, chunk):
  row_iota = lax.iota(jnp.int32, L_)

  @pl.loop(0, NBATCH_)
  def _(t):
    r0 = t * L_
    for e in range(L_):
      row = r0 + e
      def _bf(buf, k):
        return plsc.bitcast(buf[row, pl.ds(k * L_, L_)], jnp.bfloat16)

      p = _bf(u_buf, 0) * _bf(v_buf, 0)
      for k in range(1, D32_ // L_):
        p += _bf(u_buf, k) * _bf(v_buf, k)
      pa, pb = plsc.unpack(p, format=plsc.PackFormat.INTERLEAVED)
      acc[e, :] = pa + pb
    tot = plsc.load_gather(acc, [row_iota, jnp.zeros((L_,), jnp.int32)])
    for c in range(1, L_):
      tot += plsc.load_gather(acc, [row_iota, jnp.full((L_,), c, jnp.int32)])
    scores[pl.ds(chunk * W_ + r0, L_)] = tot


def _sc_body(h_hbm, edge_hbm, out_hbm,
             sidx, didx, ubufs, vbufs, acc, scores, h_spm, gsems):
  wid = lax.axis_index("s") * NC_ + lax.axis_index("c")
  sid = lax.axis_index("s")
  base = pl.multiple_of(wid * PER_W_, 8)

  rows_per_tile = N_NODES_ // NS_
  tbase = pl.multiple_of(sid * rows_per_tile, 8)
  pltpu.sync_copy(h_hbm.at[pl.ds(tbase, rows_per_tile)],
                  h_spm.at[pl.ds(tbase, rows_per_tile)])

  pltpu.sync_copy(edge_hbm.at[0, pl.ds(base, PER_W_)], sidx)
  pltpu.sync_copy(edge_hbm.at[1, pl.ds(base, PER_W_)], didx)

  plsc.subcore_barrier()

  for b in range(NSLOT_):
    _issue_gathers(h_spm, sidx, didx, ubufs[b], vbufs[b], gsems[b], b)

  @pl.loop(0, NCHUNK_ // NSLOT_ - 1)
  def _(i):
    for b in range(NSLOT_):
      chunk = i * NSLOT_ + b
      _wait_gathers(h_spm, sidx, didx, ubufs[b], vbufs[b], gsems[b], chunk)
      _compute_chunk(ubufs[b], vbufs[b], acc, scores, chunk)
      _issue_gathers(h_spm, sidx, didx, ubufs[b], vbufs[b], gsems[b],
                     chunk + NSLOT_)

  for b in range(NSLOT_):
    chunk = NCHUNK_ - NSLOT_ + b
    _wait_gathers(h_spm, sidx, didx, ubufs[b], vbufs[b], gsems[b], chunk)
    _compute_chunk(ubufs[b], vbufs[b], acc, scores, chunk)

  pltpu.sync_copy(scores, out_hbm.at[pl.ds(base, PER_W_)])


@jax.jit
def _score_sc(h, edge_index):
  mesh = plsc.VectorSubcoreMesh(core_axis_name="c", subcore_axis_name="s")
  cp = pltpu.CompilerParams()
  if "needs_layout_passes" in pltpu.CompilerParams.__dataclass_fields__:
    cp = dataclasses.replace(cp, needs_layout_passes=False)
  if "use_tc_tiling_on_sc" in pltpu.CompilerParams.__dataclass_fields__:
    cp = dataclasses.replace(cp, use_tc_tiling_on_sc=False)
  kfn = pl.kernel(
      _sc_body,
      out_type=jax.ShapeDtypeStruct((N_EDGES_,), jnp.float32),
      mesh=mesh,
      scratch_types=[
          pltpu.VMEM((PER_W_,), jnp.int32),
          pltpu.VMEM((PER_W_,), jnp.int32),
          [pltpu.VMEM((W_, D32_), jnp.int32) for _ in range(NSLOT_)],
          [pltpu.VMEM((W_, D32_), jnp.int32) for _ in range(NSLOT_)],
          pltpu.VMEM((L_, L_), jnp.float32),
          pltpu.VMEM((PER_W_,), jnp.float32),
          pltpu.VMEM_SHARED((N_NODES_, D32_), jnp.int32),
          [pltpu.SemaphoreType.DMA for _ in range(NSLOT_)],
      ],
      compiler_params=cp,
  )
  return kfn(h, edge_index)


def kernel(h, edge_index):
  ui = lax.bitcast_convert_type(h, jnp.uint32)
  r = (ui + jnp.uint32(0x7FFF) + ((ui >> 16) & jnp.uint32(1))) >> 16
  h32 = lax.bitcast_convert_type(
      r[:, :D32_] | (r[:, D32_:] << 16), jnp.int32)
  scores = _score_sc(h32, edge_index.astype(jnp.int32))
  return scores.reshape(N_EDGES_, 1)


# speedup vs baseline: 1.5038x; 1.5038x over previous
import dataclasses

import jax
import jax.numpy as jnp
from jax import lax
from jax.experimental import pallas as pl
from jax.experimental.pallas import tpu as pltpu
from jax.experimental.pallas import tpu_sc as plsc

N_NODES_ = 10000
N_EDGES_ = 320000
D_ = 128

NC_ = 2
NS_ = 16
L_ = 16
NW_ = NC_ * NS_
PER_W_ = N_EDGES_ // NW_
W_ = 400
NCHUNK_ = PER_W_ // W_
NBATCH_ = W_ // L_
D32_ = D_ // 2


def _issue_gathers(h_hbm, sidx, didx, u_buf, v_buf, sem, chunk):
  off = pl.multiple_of(chunk * W_, 8)
  pltpu.make_async_copy(h_hbm.at[sidx.at[pl.ds(off, W_)]], u_buf, sem).start()
  pltpu.make_async_copy(h_hbm.at[didx.at[pl.ds(off, W_)]], v_buf, sem).start()


def _wait_gathers(h_hbm, sidx, didx, u_buf, v_buf, sem, chunk):
  off = pl.multiple_of(chunk * W_, 8)
  pltpu.make_async_copy(h_hbm.at[sidx.at[pl.ds(off, W_)]], u_buf, sem).wait()
  pltpu.make_async_copy(h_hbm.at[didx.at[pl.ds(off, W_)]], v_buf, sem).wait()


def _compute_chunk(u_buf, v_buf, acc, s_buf):
  row_iota = lax.iota(jnp.int32, L_)

  @plsc.parallel_loop(0, NBATCH_)
  def _(t):
    r0 = t * L_
    for e in range(L_):
      row = r0 + e
      def _bf(buf, k):
        return plsc.bitcast(buf[row, pl.ds(k * L_, L_)], jnp.bfloat16)

      p = _bf(u_buf, 0) * _bf(v_buf, 0)
      for k in range(1, D32_ // L_):
        p += _bf(u_buf, k) * _bf(v_buf, k)
      pa, pb = plsc.unpack(p, format=plsc.PackFormat.INTERLEAVED)
      acc[row, :] = pa + pb
    ridx = row_iota + r0
    tot = plsc.load_gather(acc, [ridx, jnp.zeros((L_,), jnp.int32)])
    for c in range(1, L_):
      tot += plsc.load_gather(acc, [ridx, jnp.full((L_,), c, jnp.int32)])
    s_buf[pl.ds(r0, L_)] = tot


def _sc_body(h_hbm, edge_hbm, out_hbm,
             sidx, didx, u0, u1, v0, v1, acc, s0, s1, sem0, sem1, ssem0, ssem1):
  wid = lax.axis_index("s") * NC_ + lax.axis_index("c")
  base = pl.multiple_of(wid * PER_W_, 8)

  pltpu.sync_copy(edge_hbm.at[0, pl.ds(base, PER_W_)], sidx)
  pltpu.sync_copy(edge_hbm.at[1, pl.ds(base, PER_W_)], didx)

  ubufs = (u0, u1)
  vbufs = (v0, v1)
  sbufs = (s0, s1)
  gsems = (sem0, sem1)
  ssems = (ssem0, ssem1)

  def _store(b, chunk):
    off = pl.multiple_of(base + chunk * W_, 8)
    return pltpu.make_async_copy(sbufs[b], out_hbm.at[pl.ds(off, W_)],
                                 ssems[b])

  _issue_gathers(h_hbm, sidx, didx, u0, v0, sem0, 0)
  _issue_gathers(h_hbm, sidx, didx, u1, v1, sem1, 1)

  @pl.loop(0, NCHUNK_ // 2)
  def _(i):
    for b in range(2):
      chunk = i * 2 + b
      _wait_gathers(h_hbm, sidx, didx, ubufs[b], vbufs[b], gsems[b], chunk)

      @pl.when(chunk >= 2)
      def _():
        _store(b, chunk - 2).wait()

      _compute_chunk(ubufs[b], vbufs[b], acc, sbufs[b])
      _store(b, chunk).start()

      @pl.when(chunk + 2 < NCHUNK_)
      def _():
        _issue_gathers(h_hbm, sidx, didx, ubufs[b], vbufs[b], gsems[b],
                       chunk + 2)

  last = NCHUNK_ - 1
  _wait_gathers(h_hbm, sidx, didx, u0, v0, sem0, last)
  _store(0, last - 2).wait()
  _compute_chunk(u0, v0, acc, s0)
  _store(0, last).start()

  _store(1, last - 1).wait()
  _store(0, last).wait()


@jax.jit
def _score_sc(h, edge_index):
  mesh = plsc.VectorSubcoreMesh(core_axis_name="c", subcore_axis_name="s")
  cp = pltpu.CompilerParams()
  if "needs_layout_passes" in pltpu.CompilerParams.__dataclass_fields__:
    cp = dataclasses.replace(cp, needs_layout_passes=False)
  if "use_tc_tiling_on_sc" in pltpu.CompilerParams.__dataclass_fields__:
    cp = dataclasses.replace(cp, use_tc_tiling_on_sc=False)
  kfn = pl.kernel(
      _sc_body,
      out_type=jax.ShapeDtypeStruct((N_EDGES_,), jnp.float32),
      mesh=mesh,
      scratch_types=[
          pltpu.VMEM((PER_W_,), jnp.int32),
          pltpu.VMEM((PER_W_,), jnp.int32),
          pltpu.VMEM((W_, D32_), jnp.int32),
          pltpu.VMEM((W_, D32_), jnp.int32),
          pltpu.VMEM((W_, D32_), jnp.int32),
          pltpu.VMEM((W_, D32_), jnp.int32),
          pltpu.VMEM((W_, L_), jnp.float32),
          pltpu.VMEM((W_,), jnp.float32),
          pltpu.VMEM((W_,), jnp.float32),
          pltpu.SemaphoreType.DMA,
          pltpu.SemaphoreType.DMA,
          pltpu.SemaphoreType.DMA,
          pltpu.SemaphoreType.DMA,
      ],
      compiler_params=cp,
  )
  return kfn(h, edge_index)


def kernel(h, edge_index):
  ui = lax.bitcast_convert_type(h, jnp.uint32)
  r = (ui + jnp.uint32(0x7FFF) + ((ui >> 16) & jnp.uint32(1))) >> 16
  h32 = lax.bitcast_convert_type(
      r[:, :D32_] | (r[:, D32_:] << 16), jnp.int32)
  scores = _score_sc(h32, edge_index.astype(jnp.int32))
  return scores.reshape(N_EDGES_, 1)
